# baseline (speedup 1.0000x reference)
import functools

import jax
import jax.numpy as jnp
from jax import lax
from jax.experimental import pallas as pl
from jax.experimental.pallas import tpu as pltpu
from jax.experimental.pallas import tpu_sc as plsc

EMBED = 64
SCALE = 8.0
NC = 2
NS = 16
NW = NC * NS
CHUNK = 128
LANES = 16


@functools.partial(jax.jit, static_argnames=("n_chunks",))
def _emb_lookup(tok3, table, n_chunks):
    total = NW * n_chunks * CHUNK
    mesh = plsc.VectorSubcoreMesh(core_axis_name="c", subcore_axis_name="s")

    @functools.partial(
        pl.kernel,
        mesh=mesh,
        out_type=jax.ShapeDtypeStruct((total, EMBED), jnp.float32),
        scratch_types=[
            pltpu.VMEM((n_chunks, CHUNK), jnp.int32),
            pltpu.VMEM((CHUNK, EMBED), jnp.float32),
            pltpu.SemaphoreType.DMA,
        ],
        compiler_params=pltpu.CompilerParams(use_tc_tiling_on_sc=False),
    )
    def body(tok_hbm, table_hbm, out_hbm, idx_v, rows_v, sem):
        wid = lax.axis_index("s") * NC + lax.axis_index("c")
        base = wid * (n_chunks * CHUNK)
        pltpu.sync_copy(tok_hbm.at[wid], idx_v)

        def chunk_body(j, carry):
            pltpu.async_copy(table_hbm.at[idx_v.at[j]], rows_v, sem).wait()

            def scale_row(r, c):
                for d in range(EMBED // LANES):
                    sl = pl.ds(d * LANES, LANES)
                    rows_v[r, sl] = rows_v[r, sl] * SCALE
                return c

            lax.fori_loop(0, CHUNK, scale_row, 0)
            pltpu.sync_copy(rows_v, out_hbm.at[pl.ds(base + j * CHUNK, CHUNK)])
            return carry

        lax.fori_loop(0, n_chunks, chunk_body, 0)

    return body(tok3, table)


def kernel(tokens, table):
    b, s = tokens.shape
    total = b * s
    n_chunks = total // (NW * CHUNK)
    tok3 = tokens.astype(jnp.int32).reshape(NW, n_chunks, CHUNK)
    out = _emb_lookup(tok3, table, n_chunks)
    return out.reshape(b, s, EMBED)


# speedup vs baseline: 12.1571x; 12.1571x over previous
import functools

import jax
import jax.numpy as jnp
from jax import lax
from jax.experimental import pallas as pl
from jax.experimental.pallas import tpu as pltpu
from jax.experimental.pallas import tpu_sc as plsc

EMBED = 64
SCALE = 8.0
NC = 2
NS = 16
NW = NC * NS
CHUNK = 128
LANES = 16


@functools.partial(jax.jit, static_argnames=("n_chunks",))
def _emb_lookup(tok3, table, n_chunks):
    total = NW * n_chunks * CHUNK
    mesh = plsc.VectorSubcoreMesh(core_axis_name="c", subcore_axis_name="s")

    @functools.partial(
        pl.kernel,
        mesh=mesh,
        out_type=jax.ShapeDtypeStruct((total, EMBED), jnp.float32),
        scratch_types=[
            pltpu.VMEM((n_chunks, CHUNK), jnp.int32),
            pltpu.VMEM((2, CHUNK, EMBED), jnp.float32),
            pltpu.SemaphoreType.DMA,
            pltpu.SemaphoreType.DMA,
            pltpu.SemaphoreType.DMA,
            pltpu.SemaphoreType.DMA,
        ],
        compiler_params=pltpu.CompilerParams(use_tc_tiling_on_sc=False),
    )
    def body(tok_hbm, table_hbm, out_hbm, idx_v, rows_v, g0, g1, w0, w1):
        gsem = (g0, g1)
        wsem = (w0, w1)
        wid = lax.axis_index("s") * NC + lax.axis_index("c")
        base = wid * (n_chunks * CHUNK)
        pltpu.sync_copy(tok_hbm.at[wid], idx_v)

        pltpu.async_copy(table_hbm.at[idx_v.at[0]], rows_v.at[0], gsem[0])

        @pl.loop(0, n_chunks, step=2)
        def outer(j0):
            for b in range(2):
                j = j0 + b
                other = 1 - b

                @pl.when(j + 1 < n_chunks)
                def _():
                    @pl.when(j >= 1)
                    def _():
                        pltpu.make_async_copy(
                            rows_v.at[other],
                            out_hbm.at[pl.ds(0, CHUNK)],
                            wsem[other],
                        ).wait()

                    pltpu.async_copy(
                        table_hbm.at[idx_v.at[j + 1]], rows_v.at[other],
                        gsem[other],
                    )

                pltpu.make_async_copy(
                    table_hbm.at[pl.ds(0, CHUNK)], rows_v.at[b], gsem[b]
                ).wait()

                @plsc.parallel_loop(0, CHUNK, 1, unroll=8)
                def scale_row(r):
                    for d in range(EMBED // LANES):
                        sl = pl.ds(d * LANES, LANES)
                        rows_v[b, r, sl] = rows_v[b, r, sl] * SCALE

                pltpu.async_copy(
                    rows_v.at[b],
                    out_hbm.at[pl.ds(base + j * CHUNK, CHUNK)],
                    wsem[b],
                )

        for b in range(2):
            pltpu.make_async_copy(
                rows_v.at[b], out_hbm.at[pl.ds(0, CHUNK)], wsem[b]
            ).wait()

    return body(tok3, table)


def kernel(tokens, table):
    b, s = tokens.shape
    total = b * s
    n_chunks = total // (NW * CHUNK)
    tok3 = tokens.astype(jnp.int32).reshape(NW, n_chunks, CHUNK)
    out = _emb_lookup(tok3, table, n_chunks)
    return out.reshape(b, s, EMBED)
